# baseline (speedup 1.0000x reference)
import functools

import jax
import jax.numpy as jnp
from jax import lax
from jax.experimental import pallas as pl
from jax.experimental.pallas import tpu as pltpu
from jax.experimental.pallas import tpu_sc as plsc

VOCAB_N = 1000000
EMB_D = 64
OUT_D = 16
SEQ = 200
BATCH_N = 4096

NUM_CORES = 2
NUM_SUBCORES = 16
NW = NUM_CORES * NUM_SUBCORES
COLS_PER_W = BATCH_N // NW
CHUNK_COLS = 16
N_CHUNKS = COLS_PER_W // CHUNK_COLS
SEQ_PAD = 208
ROWS_PER_CHUNK = CHUNK_COLS * SEQ_PAD

SLAB = 131072
PROJ_BLK = 2048
N_PROJ_STEPS = SLAB // PROJ_BLK
TAB_ROWS = 8 * SLAB
LAST_EMB_BLK = (VOCAB_N - 1) // PROJ_BLK


def _proj_body(*refs):
    wt8_ref = refs[8]
    out_ref = refs[9]
    stacked = jnp.concatenate([refs[k][...] for k in range(8)], axis=0)
    out_ref[...] = jax.lax.dot_general(
        stacked,
        wt8_ref[...],
        dimension_numbers=(((0,), (0,)), ((), ())),
        preferred_element_type=jnp.float32,
    )


def _slab_spec(k):
    return pl.BlockSpec(
        (EMB_D, PROJ_BLK),
        lambda i, k=k: (0, jnp.minimum(N_PROJ_STEPS * k + i, LAST_EMB_BLK)),
    )


@jax.jit
def _project(embt, wt8):
    packed = pl.pallas_call(
        _proj_body,
        grid=(N_PROJ_STEPS,),
        in_specs=[_slab_spec(k) for k in range(8)]
        + [pl.BlockSpec((8 * EMB_D, 128), lambda i: (0, 0))],
        out_specs=pl.BlockSpec((PROJ_BLK, 128), lambda i: (i, 0)),
        out_shape=jax.ShapeDtypeStruct((SLAB, 128), jnp.float32),
    )(*([embt] * 8), wt8)
    return packed.reshape(TAB_ROWS, OUT_D)


def _sc_body(
    text_hbm, tab_hbm, b_hbm, out_hbm,
    idx0, idx1, rows0, rows1, st0, st1, b_v, sem0, sem1,
):
    wid = lax.axis_index("s") * NUM_CORES + lax.axis_index("c")
    pltpu.sync_copy(b_hbm, b_v)
    bias = b_v[...]
    idx = (idx0, idx1)
    rows = (rows0, rows1)
    st = (st0, st1)
    sem = (sem0, sem1)

    def fire(g, buf):
        pltpu.sync_copy(text_hbm.at[wid * N_CHUNKS + g], idx[buf])
        return pltpu.async_copy(tab_hbm.at[idx[buf]], rows[buf], sem[buf])

    def drain(g, buf, cp):
        cp.wait()
        rv = rows[buf]

        def col_body(c, _):
            base = c * SEQ_PAD
            z = jnp.zeros((OUT_D,), jnp.float32)

            @plsc.parallel_loop(0, SEQ_PAD, 8, unroll=2, carry=(z,) * 8)
            def accs(r, a):
                i = base + r
                return tuple(a[j] + rv[i + j] for j in range(8))

            s = accs
            s4 = (s[0] + s[1], s[2] + s[3], s[4] + s[5], s[6] + s[7])
            st[buf][c] = ((s4[0] + s4[1]) + (s4[2] + s4[3])) + bias
            return 0

        pass
        pltpu.sync_copy(
            st[buf],
            out_hbm.at[pl.ds(wid * COLS_PER_W + g * CHUNK_COLS, CHUNK_COLS)],
        )

    cp = fire(0, 0)
    for g in range(N_CHUNKS):
        nxt = None
        if g + 1 < N_CHUNKS:
            nxt = fire(g + 1, (g + 1) % 2)
        drain(g, g % 2, cp)
        cp = nxt


@jax.jit
def _sc_pool(text3, table, b):
    mesh = plsc.VectorSubcoreMesh(
        core_axis_name="c", subcore_axis_name="s", num_cores=NUM_CORES
    )
    run = functools.partial(
        pl.kernel,
        out_type=jax.ShapeDtypeStruct((BATCH_N, OUT_D), jnp.float32),
        mesh=mesh,
        scratch_types=[
            pltpu.VMEM((ROWS_PER_CHUNK,), jnp.int32),
            pltpu.VMEM((ROWS_PER_CHUNK,), jnp.int32),
            pltpu.VMEM((ROWS_PER_CHUNK, OUT_D), jnp.float32),
            pltpu.VMEM((ROWS_PER_CHUNK, OUT_D), jnp.float32),
            pltpu.VMEM((CHUNK_COLS, OUT_D), jnp.float32),
            pltpu.VMEM((CHUNK_COLS, OUT_D), jnp.float32),
            pltpu.VMEM((OUT_D,), jnp.float32),
            pltpu.SemaphoreType.DMA,
            pltpu.SemaphoreType.DMA,
        ],
        compiler_params=pltpu.CompilerParams(use_tc_tiling_on_sc=False),
    )(_sc_body)
    return run(text3, table, b)


def kernel(text, emb, W, b):
    wt = (W.T * (1.0 / SEQ)).astype(jnp.float32)
    wt8 = jax.scipy.linalg.block_diag(*([wt] * 8))
    table = _project(emb.T, wt8)
    ti = text.astype(jnp.int32)
    ti = (ti & (SLAB - 1)) * 8 + (ti >> 17)
    textT = jnp.transpose(ti)
    textp = jnp.pad(textT, ((0, 0), (0, SEQ_PAD - SEQ)))
    text3 = textp.reshape(BATCH_N // CHUNK_COLS, ROWS_PER_CHUNK)
    return _sc_pool(text3, table, b)


# speedup vs baseline: 1.8067x; 1.8067x over previous
import functools

import jax
import jax.numpy as jnp
from jax import lax
from jax.experimental import pallas as pl
from jax.experimental.pallas import tpu as pltpu
from jax.experimental.pallas import tpu_sc as plsc

VOCAB_N = 1000000
EMB_D = 64
OUT_D = 16
SEQ = 200
BATCH_N = 4096

NUM_CORES = 2
NUM_SUBCORES = 16
NW = NUM_CORES * NUM_SUBCORES
COLS_PER_W = BATCH_N // NW
CHUNK_COLS = 16
N_CHUNKS = COLS_PER_W // CHUNK_COLS
SEQ_PAD = 200
ROWS_PER_CHUNK = CHUNK_COLS * SEQ_PAD

SLAB = 131072
PROJ_BLK = 2048
N_PROJ_STEPS = SLAB // PROJ_BLK
TAB_ROWS = 8 * SLAB
LAST_EMB_BLK = (VOCAB_N - 1) // PROJ_BLK


def _proj_body(*refs):
    wt8_ref = refs[8]
    out_ref = refs[9]
    stacked = jnp.concatenate([refs[k][...] for k in range(8)], axis=0)
    out_ref[...] = jax.lax.dot_general(
        stacked,
        wt8_ref[...],
        dimension_numbers=(((0,), (0,)), ((), ())),
        preferred_element_type=jnp.float32,
    )


def _slab_spec(k):
    return pl.BlockSpec(
        (EMB_D, PROJ_BLK),
        lambda i, k=k: (0, jnp.minimum(N_PROJ_STEPS * k + i, LAST_EMB_BLK)),
    )


@jax.jit
def _project(embt, wt8):
    packed = pl.pallas_call(
        _proj_body,
        grid=(N_PROJ_STEPS,),
        in_specs=[_slab_spec(k) for k in range(8)]
        + [pl.BlockSpec((8 * EMB_D, 128), lambda i: (0, 0))],
        out_specs=pl.BlockSpec((PROJ_BLK, 128), lambda i: (i, 0)),
        out_shape=jax.ShapeDtypeStruct((SLAB, 128), jnp.float32),
    )(*([embt] * 8), wt8)
    return packed.reshape(TAB_ROWS, OUT_D)


def _sc_body(
    text_hbm, tab_hbm, b_hbm, out_hbm,
    idx0, idx1, rows0, rows1, st0, st1, b_v, sem0, sem1,
):
    wid = lax.axis_index("s") * NUM_CORES + lax.axis_index("c")
    pltpu.sync_copy(b_hbm, b_v)
    bias = b_v[...]
    idx = (idx0, idx1)
    rows = (rows0, rows1)
    st = (st0, st1)
    sem = (sem0, sem1)

    def fire(g, buf):
        pltpu.sync_copy(text_hbm.at[wid * N_CHUNKS + g], idx[buf])
        return pltpu.async_copy(tab_hbm.at[idx[buf]], rows[buf], sem[buf])

    def drain(g, buf, cp):
        cp.wait()
        rv = rows[buf]

        def col_body(c, _):
            base = c * SEQ_PAD
            z = jnp.zeros((OUT_D,), jnp.float32)

            @plsc.parallel_loop(0, SEQ_PAD, 8, unroll=5, carry=(z,) * 8)
            def accs(r, a):
                i = base + r
                return tuple(a[j] + rv[i + j] for j in range(8))

            s = accs
            s4 = (s[0] + s[1], s[2] + s[3], s[4] + s[5], s[6] + s[7])
            st[buf][c] = ((s4[0] + s4[1]) + (s4[2] + s4[3])) + bias
            return 0

        lax.fori_loop(0, CHUNK_COLS, col_body, 0)
        pltpu.sync_copy(
            st[buf],
            out_hbm.at[pl.ds(wid * COLS_PER_W + g * CHUNK_COLS, CHUNK_COLS)],
        )

    cp = fire(0, 0)
    for g in range(N_CHUNKS):
        nxt = None
        if g + 1 < N_CHUNKS:
            nxt = fire(g + 1, (g + 1) % 2)
        drain(g, g % 2, cp)
        cp = nxt


@jax.jit
def _sc_pool(text3, table, b):
    mesh = plsc.VectorSubcoreMesh(
        core_axis_name="c", subcore_axis_name="s", num_cores=NUM_CORES
    )
    run = functools.partial(
        pl.kernel,
        out_type=jax.ShapeDtypeStruct((BATCH_N, OUT_D), jnp.float32),
        mesh=mesh,
        scratch_types=[
            pltpu.VMEM((ROWS_PER_CHUNK,), jnp.int32),
            pltpu.VMEM((ROWS_PER_CHUNK,), jnp.int32),
            pltpu.VMEM((ROWS_PER_CHUNK, OUT_D), jnp.float32),
            pltpu.VMEM((ROWS_PER_CHUNK, OUT_D), jnp.float32),
            pltpu.VMEM((CHUNK_COLS, OUT_D), jnp.float32),
            pltpu.VMEM((CHUNK_COLS, OUT_D), jnp.float32),
            pltpu.VMEM((OUT_D,), jnp.float32),
            pltpu.SemaphoreType.DMA,
            pltpu.SemaphoreType.DMA,
        ],
        compiler_params=pltpu.CompilerParams(use_tc_tiling_on_sc=False),
    )(_sc_body)
    return run(text3, table, b)


def kernel(text, emb, W, b):
    wt = (W.T * (1.0 / SEQ)).astype(jnp.float32)
    wt8 = jax.scipy.linalg.block_diag(*([wt] * 8))
    table = _project(emb.T, wt8)
    ti = text.astype(jnp.int32)
    ti = (ti & (SLAB - 1)) * 8 + (ti >> 17)
    textT = jnp.transpose(ti)
    text3 = textT.reshape(BATCH_N // CHUNK_COLS, ROWS_PER_CHUNK)
    return _sc_pool(text3, table, b)


# speedup vs baseline: 1.9090x; 1.0566x over previous
import functools

import jax
import jax.numpy as jnp
from jax import lax
from jax.experimental import pallas as pl
from jax.experimental.pallas import tpu as pltpu
from jax.experimental.pallas import tpu_sc as plsc

VOCAB_N = 1000000
EMB_D = 64
OUT_D = 16
SEQ = 200
BATCH_N = 4096

NUM_CORES = 2
NUM_SUBCORES = 16
NW = NUM_CORES * NUM_SUBCORES
COLS_PER_W = BATCH_N // NW
CHUNK_COLS = 16
N_CHUNKS = COLS_PER_W // CHUNK_COLS
SEQ_PAD = 200
ROWS_PER_CHUNK = CHUNK_COLS * SEQ_PAD

SLAB = 131072
PROJ_BLK = 2048
N_PROJ_STEPS = SLAB // PROJ_BLK
TAB_ROWS = 8 * SLAB
LAST_EMB_BLK = (VOCAB_N - 1) // PROJ_BLK


def _proj_body(*refs):
    wt8t_ref = refs[8]
    out_ref = refs[9]
    stacked = jnp.concatenate([refs[k][...] for k in range(8)], axis=0)
    qt = jax.lax.dot_general(
        wt8t_ref[...],
        stacked,
        dimension_numbers=(((1,), (0,)), ((), ())),
        preferred_element_type=jnp.float32,
    )
    out_ref[...] = qt.T


def _slab_spec(k):
    return pl.BlockSpec(
        (EMB_D, PROJ_BLK),
        lambda i, k=k: (0, jnp.minimum(N_PROJ_STEPS * k + i, LAST_EMB_BLK)),
    )


@jax.jit
def _project(embt, wt8):
    packed = pl.pallas_call(
        _proj_body,
        grid=(N_PROJ_STEPS,),
        in_specs=[_slab_spec(k) for k in range(8)]
        + [pl.BlockSpec((128, 8 * EMB_D), lambda i: (0, 0))],
        out_specs=pl.BlockSpec((PROJ_BLK, 128), lambda i: (i, 0)),
        out_shape=jax.ShapeDtypeStruct((SLAB, 128), jnp.float32),
    )(*([embt] * 8), wt8)
    return packed.reshape(TAB_ROWS, OUT_D)


def _sc_body(
    text_hbm, tab_hbm, b_hbm, out_hbm,
    idx0, idx1, rows0, rows1, st0, st1, b_v, sem0, sem1,
):
    wid = lax.axis_index("s") * NUM_CORES + lax.axis_index("c")
    pltpu.sync_copy(b_hbm, b_v)
    bias = b_v[...]
    idx = (idx0, idx1)
    rows = (rows0, rows1)
    st = (st0, st1)
    sem = (sem0, sem1)

    def fire(g, buf):
        pltpu.sync_copy(text_hbm.at[wid * N_CHUNKS + g], idx[buf])
        return pltpu.async_copy(tab_hbm.at[idx[buf]], rows[buf], sem[buf])

    def drain(g, buf, cp):
        cp.wait()
        rv = rows[buf]

        def col_body(c, _):
            base = c * SEQ_PAD
            z = jnp.zeros((OUT_D,), jnp.float32)

            @plsc.parallel_loop(0, SEQ_PAD, 8, unroll=5, carry=(z,) * 8)
            def accs(r, a):
                i = base + r
                return tuple(a[j] + rv[i + j] for j in range(8))

            s = accs
            s4 = (s[0] + s[1], s[2] + s[3], s[4] + s[5], s[6] + s[7])
            st[buf][c] = ((s4[0] + s4[1]) + (s4[2] + s4[3])) + bias
            return 0

        lax.fori_loop(0, CHUNK_COLS, col_body, 0)
        pltpu.sync_copy(
            st[buf],
            out_hbm.at[pl.ds(wid * COLS_PER_W + g * CHUNK_COLS, CHUNK_COLS)],
        )

    cp = fire(0, 0)
    for g in range(N_CHUNKS):
        nxt = None
        if g + 1 < N_CHUNKS:
            nxt = fire(g + 1, (g + 1) % 2)
        drain(g, g % 2, cp)
        cp = nxt


@jax.jit
def _sc_pool(text3, table, b):
    mesh = plsc.VectorSubcoreMesh(
        core_axis_name="c", subcore_axis_name="s", num_cores=NUM_CORES
    )
    run = functools.partial(
        pl.kernel,
        out_type=jax.ShapeDtypeStruct((BATCH_N, OUT_D), jnp.float32),
        mesh=mesh,
        scratch_types=[
            pltpu.VMEM((ROWS_PER_CHUNK,), jnp.int32),
            pltpu.VMEM((ROWS_PER_CHUNK,), jnp.int32),
            pltpu.VMEM((ROWS_PER_CHUNK, OUT_D), jnp.float32),
            pltpu.VMEM((ROWS_PER_CHUNK, OUT_D), jnp.float32),
            pltpu.VMEM((CHUNK_COLS, OUT_D), jnp.float32),
            pltpu.VMEM((CHUNK_COLS, OUT_D), jnp.float32),
            pltpu.VMEM((OUT_D,), jnp.float32),
            pltpu.SemaphoreType.DMA,
            pltpu.SemaphoreType.DMA,
        ],
        compiler_params=pltpu.CompilerParams(use_tc_tiling_on_sc=False),
    )(_sc_body)
    return run(text3, table, b)


def kernel(text, emb, W, b):
    wt = (W.T * (1.0 / SEQ)).astype(jnp.float32)
    wt8 = jax.scipy.linalg.block_diag(*([wt] * 8))
    table = _project(emb.T, wt8.T)
    ti = text.astype(jnp.int32)
    ti = (ti & (SLAB - 1)) * 8 + (ti >> 17)
    textT = jnp.transpose(ti)
    text3 = textT.reshape(BATCH_N // CHUNK_COLS, ROWS_PER_CHUNK)
    return _sc_pool(text3, table, b)
